# baseline (speedup 1.0000x reference)
import jax
import jax.numpy as jnp
from jax.experimental import pallas as pl
from jax.experimental.pallas import tpu as pltpu

N = 100000
IN_DIM = 256
HIDDEN = 512
K = 10000
R = 1024
GRID = 98
NPAD = R * GRID

NEG_INF = float("-inf")


def _mlp_body(x_ref, w1_ref, b1_ref, w2_ref, b2_ref,
              logits_ref, m_ref, s_ref, m_acc, s_acc):
    i = pl.program_id(0)
    x = x_ref[...]
    h = jnp.dot(x, w1_ref[...], preferred_element_type=jnp.float32)
    h = jnp.maximum(h + b1_ref[...], 0.0)
    hw = h * w2_ref[...]
    logit = jnp.sum(jnp.reshape(hw, (8, 128, HIDDEN)), axis=-1) + b2_ref[...]
    row = (i * R
           + jax.lax.broadcasted_iota(jnp.int32, (8, 128), 0) * 128
           + jax.lax.broadcasted_iota(jnp.int32, (8, 128), 1))
    logit = jnp.where(row < N, logit, NEG_INF)
    logits_ref[0] = logit

    @pl.when(i == 0)
    def _init():
        m_acc[...] = jnp.full((8, 128), NEG_INF, jnp.float32)
        s_acc[...] = jnp.zeros((8, 128), jnp.float32)

    m_old = m_acc[...]
    m_new = jnp.maximum(m_old, logit)
    scale = jnp.where(m_old == NEG_INF, 0.0, jnp.exp(m_old - m_new))
    term = jnp.where(logit == NEG_INF, 0.0, jnp.exp(logit - m_new))
    s_new = s_acc[...] * scale + term
    m_acc[...] = m_new
    s_acc[...] = s_new

    @pl.when(i == GRID - 1)
    def _fin():
        m_all = jnp.max(m_new)
        s_all = jnp.sum(s_new * jnp.exp(m_new - m_all))
        m_ref[...] = jnp.reshape(m_all, (1, 1))
        s_ref[...] = jnp.reshape(s_all, (1, 1))


def _mlp_logits(xp, W1, b1, W2, b2):
    return pl.pallas_call(
        _mlp_body,
        grid=(GRID,),
        in_specs=[
            pl.BlockSpec((R, IN_DIM), lambda i: (i, 0)),
            pl.BlockSpec((IN_DIM, HIDDEN), lambda i: (0, 0)),
            pl.BlockSpec((1, HIDDEN), lambda i: (0, 0)),
            pl.BlockSpec((1, HIDDEN), lambda i: (0, 0)),
            pl.BlockSpec((1, 1), lambda i: (0, 0)),
        ],
        out_specs=[
            pl.BlockSpec((1, 8, 128), lambda i: (i, 0, 0)),
            pl.BlockSpec((1, 1), lambda i: (0, 0)),
            pl.BlockSpec((1, 1), lambda i: (0, 0)),
        ],
        out_shape=[
            jax.ShapeDtypeStruct((GRID, 8, 128), jnp.float32),
            jax.ShapeDtypeStruct((1, 1), jnp.float32),
            jax.ShapeDtypeStruct((1, 1), jnp.float32),
        ],
        scratch_shapes=[
            pltpu.VMEM((8, 128), jnp.float32),
            pltpu.VMEM((8, 128), jnp.float32),
        ],
    )(xp, W1, b1, W2, b2)


def kernel(point_features, point_locations, W1, b1, W2, b2):
    xp = jnp.pad(point_features, ((0, NPAD - N), (0, 0)))
    b1r = b1.reshape(1, HIDDEN)
    w2r = W2.reshape(1, HIDDEN)
    b2r = b2.reshape(1, 1)
    logits3, m, s = _mlp_logits(xp, W1, b1r, w2r, b2r)
    logits = logits3.reshape(NPAD)[:N]
    vals, idx = jax.lax.top_k(logits, K)
    values = jnp.exp(vals - m[0, 0]) / s[0, 0]
    gathered = point_locations[idx]
    mean_location = jnp.mean(gathered, axis=0)
    highest = point_locations[idx[0]]
    return jnp.concatenate([values, mean_location, highest], axis=0)
